# baseline (speedup 1.0000x reference)
import jax
import jax.numpy as jnp
from jax import lax
from jax.experimental import pallas as pl
from jax.experimental.pallas import tpu as pltpu


def _moe_kernel(x_hbm, wg_ref, bg_ref, w1_hbm, b1_ref, w2_hbm, b2_ref,
                out_ref, xv, w1buf, w2buf, sem_x, sem_w1, sem_w2):
    BT, D = xv.shape
    E = wg_ref.shape[1]
    B, T, O = out_ref.shape
    S = 2 * B

    HD2 = D // 2

    def _w_copies(t, slot):
        return (
            pltpu.make_async_copy(w1_hbm.at[t, 0:HD2], w1buf.at[slot, 0:HD2],
                                  sem_w1.at[slot, 0]),
            pltpu.make_async_copy(w1_hbm.at[t, HD2:D], w1buf.at[slot, HD2:D],
                                  sem_w1.at[slot, 1]),
            pltpu.make_async_copy(w2_hbm.at[t, 0:HD2], w2buf.at[slot, 0:HD2],
                                  sem_w2.at[slot, 0]),
            pltpu.make_async_copy(w2_hbm.at[t, HD2:D], w2buf.at[slot, HD2:D],
                                  sem_w2.at[slot, 1]),
        )

    cx = pltpu.make_async_copy(x_hbm, xv, sem_x)
    cx.start()
    for c in _w_copies(0, 0):
        c.start()
    cx.wait()
    X = xv[...]

    logits = jnp.dot(X, wg_ref[...], preferred_element_type=jnp.float32)
    logits = logits + bg_ref[...]
    m = jnp.max(logits, axis=-1, keepdims=True)
    ex = jnp.exp(logits - m)
    scores = ex / jnp.sum(ex, axis=-1, keepdims=True)
    col = lax.broadcasted_iota(jnp.int32, scores.shape, 1)
    m1 = jnp.max(scores, axis=-1, keepdims=True)
    e1 = jnp.min(jnp.where(scores == m1, col, E), axis=-1, keepdims=True)
    rest = jnp.where(col == e1, -jnp.inf, scores)
    m2 = jnp.max(rest, axis=-1, keepdims=True)
    e2 = jnp.min(jnp.where(rest == m2, col, E), axis=-1, keepdims=True)
    sel1 = (col == e1).astype(jnp.float32)
    sel2 = (col == e2).astype(jnp.float32)
    v1 = jnp.sum(sel1 * scores, axis=-1, keepdims=True)
    v2 = jnp.sum(sel2 * scores, axis=-1, keepdims=True)
    den = jnp.maximum(v1 + v2, 1e-12)
    aux4 = jnp.concatenate(
        [e1.astype(jnp.float32), e2.astype(jnp.float32), v1 / den, v2 / den],
        axis=1)

    jb = lax.broadcasted_iota(jnp.int32, (S, BT), 0)
    jr = lax.broadcasted_iota(jnp.int32, (S, BT), 1)
    odd = lax.broadcasted_iota(jnp.int32, (S, 1), 0) % 2
    rbase = (lax.broadcasted_iota(jnp.int32, (S, 1), 0) // 2) * T
    cb = lax.broadcasted_iota(jnp.int32, (B, S), 0)
    cj = lax.broadcasted_iota(jnp.int32, (B, S), 1)
    comb = (cj // 2 == cb).astype(jnp.float32)

    for t in range(T):
        slot = t % 2
        nxt = (t + 1) % 2
        if t + 1 < T:
            for c in _w_copies(t + 1, nxt):
                c.start()
        for c in _w_copies(t, slot):
            c.wait()

        ssum = (jr == (jb // 2) * T + t).astype(jnp.float32)
        g4 = jnp.dot(ssum, aux4, preferred_element_type=jnp.float32)
        e_sel = jnp.where(odd == 0, g4[:, 0:1], g4[:, 1:2])
        w_sel = jnp.where(odd == 0, g4[:, 2:3], g4[:, 3:4])
        u = rbase + e_sel.astype(jnp.int32)

        p = (jr == u).astype(jnp.float32)
        xsel = jnp.dot(p, X, preferred_element_type=jnp.float32)

        h = jnp.dot(xsel, w1buf[slot], preferred_element_type=jnp.float32)
        h = jnp.maximum(h + b1_ref[t:t + 1, :], 0.0)
        hw = h * w_sel
        mixed = jnp.dot(comb, hw, preferred_element_type=jnp.float32)
        out = jnp.dot(mixed, w2buf[slot], preferred_element_type=jnp.float32)
        out_ref[:, t, :] = out + b2_ref[t:t + 1, :]


def kernel(x, num_experts_chosen, Wg, bg, W1, b1, W2, b2):
    del num_experts_chosen
    B, T, D = x.shape
    E, _, H = W1.shape
    O = W2.shape[2]
    BT = B * T
    x2 = x.reshape(BT, D)
    bg2 = bg.reshape(1, E)

    hbm = pl.BlockSpec(memory_space=pltpu.MemorySpace.HBM)
    out = pl.pallas_call(
        _moe_kernel,
        in_specs=[
            hbm,
            pl.BlockSpec((D, E), lambda: (0, 0)),
            pl.BlockSpec((1, E), lambda: (0, 0)),
            hbm,
            pl.BlockSpec((E, H), lambda: (0, 0)),
            hbm,
            pl.BlockSpec((E, O), lambda: (0, 0)),
        ],
        out_specs=pl.BlockSpec((B, T, O), lambda: (0, 0, 0)),
        out_shape=jax.ShapeDtypeStruct((B, T, O), jnp.float32),
        scratch_shapes=[
            pltpu.VMEM((BT, D), jnp.float32),
            pltpu.VMEM((2, D, H), jnp.float32),
            pltpu.VMEM((2, H, O), jnp.float32),
            pltpu.SemaphoreType.DMA,
            pltpu.SemaphoreType.DMA((2, 2)),
            pltpu.SemaphoreType.DMA((2, 2)),
        ],
    )(x2, Wg, bg2, W1, b1, W2, b2)
    return out


# speedup vs baseline: 1.0405x; 1.0405x over previous
import jax
import jax.numpy as jnp
from jax import lax
from jax.experimental import pallas as pl
from jax.experimental.pallas import tpu as pltpu


def _moe_kernel(x_hbm, wg_ref, bg_ref, w1_hbm, b1_ref, w2_hbm, b2_ref,
                out_ref, xv, w1buf, w2buf, sem_x, sem_w1, sem_w2):
    BT, D = xv.shape
    E = wg_ref.shape[1]
    B, T, O = out_ref.shape
    S = 2 * B

    cx = pltpu.make_async_copy(x_hbm, xv, sem_x)
    cx.start()
    pltpu.make_async_copy(w1_hbm.at[0], w1buf.at[0], sem_w1.at[0]).start()
    pltpu.make_async_copy(w2_hbm.at[0], w2buf.at[0], sem_w2.at[0]).start()
    cx.wait()
    X = xv[...]

    logits = jnp.dot(X, wg_ref[...], preferred_element_type=jnp.float32)
    logits = logits + bg_ref[...]
    m = jnp.max(logits, axis=-1, keepdims=True)
    ex = jnp.exp(logits - m)
    scores = ex / jnp.sum(ex, axis=-1, keepdims=True)
    col = lax.broadcasted_iota(jnp.int32, scores.shape, 1)
    m1 = jnp.max(scores, axis=-1, keepdims=True)
    e1 = jnp.min(jnp.where(scores == m1, col, E), axis=-1, keepdims=True)
    rest = jnp.where(col == e1, -jnp.inf, scores)
    m2 = jnp.max(rest, axis=-1, keepdims=True)
    e2 = jnp.min(jnp.where(rest == m2, col, E), axis=-1, keepdims=True)
    sel1 = (col == e1).astype(jnp.float32)
    sel2 = (col == e2).astype(jnp.float32)
    v1 = jnp.sum(sel1 * scores, axis=-1, keepdims=True)
    v2 = jnp.sum(sel2 * scores, axis=-1, keepdims=True)
    den = jnp.maximum(v1 + v2, 1e-12)
    aux4 = jnp.concatenate(
        [e1.astype(jnp.float32), e2.astype(jnp.float32), v1 / den, v2 / den],
        axis=1)

    jb = lax.broadcasted_iota(jnp.int32, (S, BT), 0)
    jr = lax.broadcasted_iota(jnp.int32, (S, BT), 1)
    odd = lax.broadcasted_iota(jnp.int32, (S, 1), 0) % 2
    rbase = (lax.broadcasted_iota(jnp.int32, (S, 1), 0) // 2) * T
    cb = lax.broadcasted_iota(jnp.int32, (B, S), 0)
    cj = lax.broadcasted_iota(jnp.int32, (B, S), 1)
    comb = (cj // 2 == cb).astype(jnp.float32)

    for t in range(T):
        slot = t % 2
        nxt = (t + 1) % 2
        if t + 1 < T:
            pltpu.make_async_copy(w1_hbm.at[t + 1], w1buf.at[nxt],
                                  sem_w1.at[nxt]).start()
            pltpu.make_async_copy(w2_hbm.at[t + 1], w2buf.at[nxt],
                                  sem_w2.at[nxt]).start()
        pltpu.make_async_copy(w1_hbm.at[t], w1buf.at[slot],
                              sem_w1.at[slot]).wait()
        pltpu.make_async_copy(w2_hbm.at[t], w2buf.at[slot],
                              sem_w2.at[slot]).wait()

        ssum = (jr == (jb // 2) * T + t).astype(jnp.float32)
        g4 = jnp.dot(ssum, aux4, preferred_element_type=jnp.float32)
        e_sel = jnp.where(odd == 0, g4[:, 0:1], g4[:, 1:2])
        w_sel = jnp.where(odd == 0, g4[:, 2:3], g4[:, 3:4])
        u = rbase + e_sel.astype(jnp.int32)

        p = (jr == u).astype(jnp.float32)
        xsel = jnp.dot(p, X, preferred_element_type=jnp.float32)

        h = jnp.dot(xsel, w1buf[slot], preferred_element_type=jnp.float32)
        h = jnp.maximum(h + b1_ref[t:t + 1, :], 0.0)
        hw = h * w_sel
        mixed = jnp.dot(comb, hw, preferred_element_type=jnp.float32)
        out = jnp.dot(mixed, w2buf[slot], preferred_element_type=jnp.float32)
        out_ref[:, t, :] = out + b2_ref[t:t + 1, :]


def kernel(x, num_experts_chosen, Wg, bg, W1, b1, W2, b2):
    del num_experts_chosen
    B, T, D = x.shape
    E, _, H = W1.shape
    O = W2.shape[2]
    BT = B * T
    x2 = x.reshape(BT, D)
    bg2 = bg.reshape(1, E)

    hbm = pl.BlockSpec(memory_space=pltpu.MemorySpace.HBM)
    out = pl.pallas_call(
        _moe_kernel,
        in_specs=[
            hbm,
            pl.BlockSpec((D, E), lambda: (0, 0)),
            pl.BlockSpec((1, E), lambda: (0, 0)),
            hbm,
            pl.BlockSpec((E, H), lambda: (0, 0)),
            hbm,
            pl.BlockSpec((E, O), lambda: (0, 0)),
        ],
        out_specs=pl.BlockSpec((B, T, O), lambda: (0, 0, 0)),
        out_shape=jax.ShapeDtypeStruct((B, T, O), jnp.float32),
        scratch_shapes=[
            pltpu.VMEM((BT, D), jnp.float32),
            pltpu.VMEM((2, D, H), jnp.float32),
            pltpu.VMEM((2, H, O), jnp.float32),
            pltpu.SemaphoreType.DMA,
            pltpu.SemaphoreType.DMA((2,)),
            pltpu.SemaphoreType.DMA((2,)),
        ],
    )(x2, Wg, bg2, W1, b1, W2, b2)
    return out
